# baseline (speedup 1.0000x reference)
import functools

import jax
import jax.numpy as jnp
from jax import lax
from jax.experimental import pallas as pl
from jax.experimental.pallas import tpu as pltpu
from jax.experimental.pallas import tpu_sc as plsc

N_AT = 10000
N_CG = 1000
APC = 10
F = 128
K = 16
E_AT = 320000
E_CG = 16000

NC = 2
NS = 16
NW = NC * NS
L = 16

EPT = E_AT // NW
CH = 64
NCHUNK = EPT // CH
NCLS = 9
SLAB = N_AT
R_ROWS = NCLS * SLAB
ROWS_PT = R_ROWS // NS
ZCH = 125
WPT = SLAB // NS

_CENTERS = [5.0 * k / (K - 1) for k in range(K)]


NPAIR = NCHUNK // 2


def _sc_edge_body(rec_h, src_h, dst_h, out_h,
                  se, de, rc0, rc1, rb0, rb1, sb0, sb1, tib,
                  smr0, smr1, sms0, sms1, smz, r_sh):
  c = lax.axis_index("c")
  s = lax.axis_index("s")
  w = s * NC + c
  lane = lax.iota(jnp.int32, L)

  def edge_group(recbuf, row0, g, doff):
    rs = g * L + lane
    rd = rs + doff
    c0 = jnp.full((L,), 0, jnp.int32)
    c1 = jnp.full((L,), 1, jnp.int32)
    c2 = jnp.full((L,), 2, jnp.int32)
    c3 = jnp.full((L,), 3, jnp.int32)
    dx = plsc.load_gather(recbuf, [rd, c0]) - plsc.load_gather(recbuf, [rs, c0])
    dy = plsc.load_gather(recbuf, [rd, c1]) - plsc.load_gather(recbuf, [rs, c1])
    dz = plsc.load_gather(recbuf, [rd, c2]) - plsc.load_gather(recbuf, [rs, c2])
    d2 = dx * dx + dy * dy + dz * dz + 1e-8
    bits = plsc.bitcast(d2, jnp.int32)
    yb = plsc.bitcast(jnp.int32(0x5F3759DF) - (bits >> 1), jnp.float32)
    for _ in range(3):
      yb = yb * (1.5 - 0.5 * d2 * yb * yb)
    dist = d2 * yb
    cls = plsc.load_gather(recbuf, [rd, c3]).astype(jnp.int32) - 1
    sv = plsc.load_gather(se, [row0 + g * L + lane])
    seg = sv * NCLS + cls
    return seg, dist

  def compute(chunk, rc, rb, sb):
    for g in range(CH // L):
      seg, dist = edge_group(rc, chunk * CH, g, CH)
      sb[pl.ds(g * L, L)] = seg
      for k in range(K):
        t = dist - _CENTERS[k]
        v = jnp.exp(t * t * (-10.0))
        col = jnp.full((L,), k, jnp.int32)
        plsc.store_scatter(rb, [g * L + lane, col], v)

  for i in range(ZCH):
    rc0[i, :] = jnp.zeros((L,), jnp.float32)
  zsrc = rc0.at[pl.ds(0, ZCH)]
  nz = ROWS_PT // ZCH
  for i in range(nz):
    pltpu.async_copy(zsrc, r_sh.at[pl.ds(s * ROWS_PT + i * ZCH, ZCH)], smz)
  pltpu.sync_copy(src_h.at[pl.ds(w * EPT, EPT)], se)
  pltpu.sync_copy(dst_h.at[pl.ds(w * EPT, EPT)], de)
  for i in range(nz):
    pltpu.make_async_copy(
        zsrc, r_sh.at[pl.ds(s * ROWS_PT + i * ZCH, ZCH)], smz).wait()
  plsc.subcore_barrier()

  def issue_gather(chunk, rc, smr):
    pltpu.async_copy(rec_h.at[se.at[pl.ds(chunk * CH, CH)]],
                     rc.at[pl.ds(0, CH)], smr)
    pltpu.async_copy(rec_h.at[de.at[pl.ds(chunk * CH, CH)]],
                     rc.at[pl.ds(CH, CH)], smr)

  def wait_gather(chunk, rc, smr):
    pltpu.make_async_copy(rec_h.at[se.at[pl.ds(chunk * CH, CH)]],
                          rc.at[pl.ds(0, CH)], smr).wait()
    pltpu.make_async_copy(rec_h.at[de.at[pl.ds(chunk * CH, CH)]],
                          rc.at[pl.ds(CH, CH)], smr).wait()

  issue_gather(0, rc0, smr0)
  issue_gather(1, rc1, smr1)

  def pair_body(j, carry):
    for (off, rc, rb, sb, smr, sms) in (
        (0, rc0, rb0, sb0, smr0, sms0),
        (1, rc1, rb1, sb1, smr1, sms1),
    ):
      ch0 = 2 * j + off
      wait_gather(ch0, rc, smr)

      @pl.when(j > 0)
      def _():
        pltpu.make_async_copy(rb, r_sh.at[sb], sms).wait()

      compute(ch0, rc, rb, sb)
      pltpu.async_copy(rb, r_sh.at[sb], sms, add=True)

      @pl.when(j < NPAIR - 1)
      def _():
        issue_gather(ch0 + 2, rc, smr)
    return carry

  lax.fori_loop(0, NPAIR, pair_body, 0)
  pltpu.make_async_copy(rb0, r_sh.at[sb0], sms0).wait()
  pltpu.make_async_copy(rb1, r_sh.at[sb1], sms1).wait()

  pltpu.sync_copy(rec_h.at[se.at[pl.ds(NCHUNK * CH, L)]],
                  rc0.at[pl.ds(0, L)])
  pltpu.sync_copy(rec_h.at[de.at[pl.ds(NCHUNK * CH, L)]],
                  rc0.at[pl.ds(L, L)])
  seg, dist = edge_group(rc0, NCHUNK * CH, 0, L)
  tib[...] = seg
  for k in range(K):
    t = dist - _CENTERS[k]
    v = jnp.exp(t * t * (-10.0))
    col = jnp.full((L,), k, jnp.int32)
    plsc.store_scatter(rb0, [lane, col], v)
  pltpu.sync_copy(rb0.at[pl.ds(0, L)], r_sh.at[tib], add=True)

  plsc.subcore_barrier()

  pltpu.sync_copy(r_sh.at[pl.ds(s * ROWS_PT, ROWS_PT)],
                  out_h.at[c, pl.ds(s * ROWS_PT, ROWS_PT)])


@functools.lru_cache(maxsize=None)
def _build_sc_kernel():
  return functools.partial(
      pl.kernel,
      out_type=jax.ShapeDtypeStruct((NC, R_ROWS, K), jnp.float32),
      mesh=plsc.VectorSubcoreMesh(core_axis_name="c", subcore_axis_name="s"),
      compiler_params=pltpu.CompilerParams(
          needs_layout_passes=False, use_tc_tiling_on_sc=False),
      scratch_types=[
          pltpu.VMEM((EPT,), jnp.int32),
          pltpu.VMEM((EPT,), jnp.int32),
          pltpu.VMEM((2 * CH, L), jnp.float32),
          pltpu.VMEM((2 * CH, L), jnp.float32),
          pltpu.VMEM((CH, L), jnp.float32),
          pltpu.VMEM((CH, L), jnp.float32),
          pltpu.VMEM((CH,), jnp.int32),
          pltpu.VMEM((CH,), jnp.int32),
          pltpu.VMEM((L,), jnp.int32),
          pltpu.SemaphoreType.DMA,
          pltpu.SemaphoreType.DMA,
          pltpu.SemaphoreType.DMA,
          pltpu.SemaphoreType.DMA,
          pltpu.SemaphoreType.DMA,
          pltpu.VMEM_SHARED((R_ROWS, K), jnp.float32),
      ],
  )(_sc_edge_body)


def _atom_body(r_ref, zv_ref, wf_ref, we_ref, wu_ref, out_ref):
  cval = (lax.broadcasted_iota(jnp.int32, (NCLS, 1, F), 0) + 1
          ).astype(jnp.float32)
  h_tab = jnp.tanh(cval * we_ref[...])
  wbig = (h_tab * wf_ref[...]).reshape(NCLS * K, F)
  rsum = r_ref[0] + r_ref[1]
  agg = jnp.dot(rsum, wbig, preferred_element_type=jnp.float32)
  h0 = jnp.tanh(zv_ref[...] * we_ref[0])
  s_i = jnp.tanh(h0 + jnp.dot(agg, wu_ref[...],
                              preferred_element_type=jnp.float32))
  out_ref[...] = jnp.mean(s_i.reshape(N_CG, APC, F), axis=1)


def _stage2(r6, zv, w_filter, w_embed, w_update):
  full = lambda shape: pl.BlockSpec(shape, lambda: tuple(0 for _ in shape))
  return pl.pallas_call(
      _atom_body,
      in_specs=[
          full((NC, N_AT, NCLS * K)),
          full((N_AT, 1)),
          full((1, K, F)),
          full((1, F)),
          full((F, F)),
      ],
      out_specs=full((N_CG, F)),
      out_shape=jax.ShapeDtypeStruct((N_CG, F), jnp.float32),
  )(r6, zv, w_filter, w_embed, w_update)


BE = 2000
NBE = E_CG // BE


def _cg_body(cic_ref, cit_ref, cjc_ref, si_ref, cgp_ref,
             wmu1_ref, wmu2_ref, wsg1_ref, wsg2_ref, wcgf_ref, wv_ref,
             mu_ref, sg_ref, rx_ref, ry_ref, rz_ref,
             accx, accy, accz):
  step = pl.program_id(0)
  bi = cic_ref[...] == lax.broadcasted_iota(jnp.int32, (BE, N_CG), 1)
  bj = cjc_ref[...] == lax.broadcasted_iota(jnp.int32, (BE, N_CG), 1)
  ohi = bi.astype(jnp.float32)
  ohj = bj.astype(jnp.float32)
  ohjh = bj.astype(jnp.bfloat16)
  ohith = (cit_ref[...].reshape(1, BE) ==
           lax.broadcasted_iota(jnp.int32, (N_CG, BE), 0)
           ).astype(jnp.bfloat16)
  cgp = cgp_ref[...]
  gpi = jnp.dot(ohi, cgp, preferred_element_type=jnp.float32)
  gpj = jnp.dot(ohj, cgp, preferred_element_type=jnp.float32)
  dux = gpj[:, 0:1] - gpi[:, 0:1]
  duy = gpj[:, 1:2] - gpi[:, 1:2]
  duz = gpj[:, 2:3] - gpi[:, 2:3]
  dn = jnp.sqrt(dux * dux + duy * duy + duz * duz + 1e-8)
  ux = dux / dn
  uy = duy / dn
  uz = duz / dn
  cent = lax.broadcasted_iota(jnp.int32, (BE, K), 1).astype(jnp.float32) * (
      5.0 / (K - 1))
  t = dn - cent
  crbf = jnp.exp(t * t * (-10.0))
  sij = jnp.dot(ohjh, si_ref[...].astype(jnp.bfloat16),
                preferred_element_type=jnp.float32)
  cm = sij * jnp.dot(crbf, wcgf_ref[...], preferred_element_type=jnp.float32)
  w = jnp.dot(cm, wv_ref[...], preferred_element_type=jnp.float32)
  scx = jnp.dot(ohith, (w * ux).astype(jnp.bfloat16),
                preferred_element_type=jnp.float32)
  scy = jnp.dot(ohith, (w * uy).astype(jnp.bfloat16),
                preferred_element_type=jnp.float32)
  scz = jnp.dot(ohith, (w * uz).astype(jnp.bfloat16),
                preferred_element_type=jnp.float32)

  @pl.when(step == 0)
  def _():
    accx[...] = scx
    accy[...] = scy
    accz[...] = scz

  @pl.when(step != 0)
  def _():
    accx[...] = accx[...] + scx
    accy[...] = accy[...] + scy
    accz[...] = accz[...] + scz

  @pl.when(step == NBE - 1)
  def _():
    si = si_ref[...]
    mu_ref[...] = jnp.dot(
        jnp.tanh(jnp.dot(si, wmu1_ref[...], preferred_element_type=jnp.float32)),
        wmu2_ref[...], preferred_element_type=jnp.float32)
    logvar = jnp.dot(
        jnp.tanh(jnp.dot(si, wsg1_ref[...], preferred_element_type=jnp.float32)),
        wsg2_ref[...], preferred_element_type=jnp.float32)
    sg_ref[...] = 1e-12 + jnp.exp(logvar * 0.5)
    chmask = (lax.broadcasted_iota(jnp.int32, (N_CG, L), 1) < APC
              ).astype(jnp.float32)
    for acc, ref, col in ((accx, rx_ref, 0), (accy, ry_ref, 1), (accz, rz_ref, 2)):
      v = acc[...]
      offs = jnp.sum(v * chmask, axis=1, keepdims=True) * (1.0 / APC)
      ref[...] = v - offs + cgp_ref[:, col:col + 1]


def _stage3(cic, cit, cjc, s_i, cgp, w_mu1, w_mu2, w_sg1, w_sg2, w_cgf, w_vp):
  full = lambda shape: pl.BlockSpec(shape, lambda i: tuple(0 for _ in shape))
  return pl.pallas_call(
      _cg_body,
      grid=(NBE,),
      in_specs=[
          pl.BlockSpec((BE, 1), lambda i: (i, 0)),
          pl.BlockSpec((1, 1, BE), lambda i: (i, 0, 0)),
          pl.BlockSpec((BE, 1), lambda i: (i, 0)),
          full((N_CG, F)),
          full((N_CG, 8)),
          full((F, F)), full((F, F)), full((F, F)), full((F, F)),
          full((K, F)), full((F, L)),
      ],
      out_specs=[
          full((N_CG, F)), full((N_CG, F)),
          full((N_CG, L)), full((N_CG, L)), full((N_CG, L)),
      ],
      out_shape=[
          jax.ShapeDtypeStruct((N_CG, F), jnp.float32),
          jax.ShapeDtypeStruct((N_CG, F), jnp.float32),
          jax.ShapeDtypeStruct((N_CG, L), jnp.float32),
          jax.ShapeDtypeStruct((N_CG, L), jnp.float32),
          jax.ShapeDtypeStruct((N_CG, L), jnp.float32),
      ],
      scratch_shapes=[
          pltpu.VMEM((N_CG, L), jnp.float32),
          pltpu.VMEM((N_CG, L), jnp.float32),
          pltpu.VMEM((N_CG, L), jnp.float32),
      ],
  )(cic, cit, cjc, s_i, cgp, w_mu1, w_mu2, w_sg1, w_sg2, w_cgf, w_vp)


def kernel(nxyz, CG_nxyz, CG_mapping, nbr_list, CG_nbr_list, num_CGs,
           W_embed, W_filter, W_update, W_mu1, W_mu2, W_sg1, W_sg2,
           W_cgf, W_cgs, W_v):
  xyz = nxyz[:, 1:]
  rec = jnp.pad(jnp.concatenate([nxyz[:, 1:4], nxyz[:, 0:1]], axis=1),
                ((0, 0), (0, L - 4)))

  r_part = _build_sc_kernel()(rec, nbr_list[:, 0], nbr_list[:, 1])
  r6 = r_part.reshape(NC, N_AT, NCLS * K)

  s_i_cg = _stage2(r6, nxyz[:, 0:1],
                   W_filter.astype(jnp.float32).reshape(1, K, F),
                   W_embed.astype(jnp.float32), W_update.astype(jnp.float32))

  cic = jnp.asarray(CG_nbr_list[:, 0]).reshape(E_CG, 1)
  cit = cic.reshape(NBE, 1, BE)
  cjc = jnp.asarray(CG_nbr_list[:, 1]).reshape(E_CG, 1)
  cgp = jnp.pad(CG_nxyz[:, 1:], ((0, 0), (0, 5)))
  w_vp = jnp.pad(W_v.astype(jnp.float32), ((0, 0), (0, L - APC)))

  mu, sigma, rx, ry, rz = _stage3(
      cic, cit, cjc, s_i_cg, cgp,
      W_mu1.astype(jnp.float32), W_mu2.astype(jnp.float32),
      W_sg1.astype(jnp.float32), W_sg2.astype(jnp.float32),
      W_cgf.astype(jnp.float32), w_vp)

  xyz_recon = jnp.stack([
      rx[:, :APC].reshape(N_AT),
      ry[:, :APC].reshape(N_AT),
      rz[:, :APC].reshape(N_AT),
  ], axis=1)
  return (mu, sigma, xyz, xyz_recon)


# speedup vs baseline: 1.0020x; 1.0020x over previous
import functools

import jax
import jax.numpy as jnp
from jax import lax
from jax.experimental import pallas as pl
from jax.experimental.pallas import tpu as pltpu
from jax.experimental.pallas import tpu_sc as plsc

N_AT = 10000
N_CG = 1000
APC = 10
F = 128
K = 16
E_AT = 320000
E_CG = 16000

NC = 2
NS = 16
NW = NC * NS
L = 16

EPT = E_AT // NW
CH = 64
NCHUNK = EPT // CH
NCLS = 9
SLAB = N_AT
R_ROWS = NCLS * SLAB
ROWS_PT = R_ROWS // NS
ZCH = 125
WPT = SLAB // NS

_CENTERS = [5.0 * k / (K - 1) for k in range(K)]


NPAIR = NCHUNK // 2


def _sc_edge_body(rec_h, src_h, dst_h, out_h,
                  se, de, rc0, rc1, rb0, rb1, sb0, sb1, tib,
                  smr0, smr1, sms0, sms1, smz, r_sh):
  c = lax.axis_index("c")
  s = lax.axis_index("s")
  w = s * NC + c
  lane = lax.iota(jnp.int32, L)

  def edge_group(recbuf, row0, g, doff):
    rs = g * L + lane
    rd = rs + doff
    c0 = jnp.full((L,), 0, jnp.int32)
    c1 = jnp.full((L,), 1, jnp.int32)
    c2 = jnp.full((L,), 2, jnp.int32)
    c3 = jnp.full((L,), 3, jnp.int32)
    dx = plsc.load_gather(recbuf, [rd, c0]) - plsc.load_gather(recbuf, [rs, c0])
    dy = plsc.load_gather(recbuf, [rd, c1]) - plsc.load_gather(recbuf, [rs, c1])
    dz = plsc.load_gather(recbuf, [rd, c2]) - plsc.load_gather(recbuf, [rs, c2])
    d2 = dx * dx + dy * dy + dz * dz + 1e-8
    bits = plsc.bitcast(d2, jnp.int32)
    yb = plsc.bitcast(jnp.int32(0x5F3759DF) - (bits >> 1), jnp.float32)
    for _ in range(3):
      yb = yb * (1.5 - 0.5 * d2 * yb * yb)
    dist = d2 * yb
    cls = plsc.load_gather(recbuf, [rd, c3]).astype(jnp.int32) - 1
    sv = plsc.load_gather(se, [row0 + g * L + lane])
    seg = sv * NCLS + cls
    return seg, dist

  def compute(chunk, rc, rb, sb):
    for g in range(CH // L):
      seg, dist = edge_group(rc, chunk * CH, g, CH)
      sb[pl.ds(g * L, L)] = seg
      for k in range(K):
        t = dist - _CENTERS[k]
        v = jnp.exp(t * t * (-10.0))
        col = jnp.full((L,), k, jnp.int32)
        plsc.store_scatter(rb, [g * L + lane, col], v)

  for i in range(ZCH):
    rc0[i, :] = jnp.zeros((L,), jnp.float32)
  zsrc = rc0.at[pl.ds(0, ZCH)]
  nz = ROWS_PT // ZCH
  for i in range(nz):
    pltpu.async_copy(zsrc, r_sh.at[pl.ds(s * ROWS_PT + i * ZCH, ZCH)], smz)
  pltpu.sync_copy(src_h.at[pl.ds(w * EPT, EPT)], se)
  pltpu.sync_copy(dst_h.at[pl.ds(w * EPT, EPT)], de)
  for i in range(nz):
    pltpu.make_async_copy(
        zsrc, r_sh.at[pl.ds(s * ROWS_PT + i * ZCH, ZCH)], smz).wait()
  plsc.subcore_barrier()

  def issue_gather(chunk, rc, smr):
    pltpu.async_copy(rec_h.at[se.at[pl.ds(chunk * CH, CH)]],
                     rc.at[pl.ds(0, CH)], smr)
    pltpu.async_copy(rec_h.at[de.at[pl.ds(chunk * CH, CH)]],
                     rc.at[pl.ds(CH, CH)], smr)

  def wait_gather(chunk, rc, smr):
    pltpu.make_async_copy(rec_h.at[se.at[pl.ds(chunk * CH, CH)]],
                          rc.at[pl.ds(0, CH)], smr).wait()
    pltpu.make_async_copy(rec_h.at[de.at[pl.ds(chunk * CH, CH)]],
                          rc.at[pl.ds(CH, CH)], smr).wait()

  issue_gather(0, rc0, smr0)
  issue_gather(1, rc1, smr1)

  def pair_body(j, carry):
    for (off, rc, rb, sb, smr, sms) in (
        (0, rc0, rb0, sb0, smr0, sms0),
        (1, rc1, rb1, sb1, smr1, sms1),
    ):
      ch0 = 2 * j + off
      wait_gather(ch0, rc, smr)

      @pl.when(j > 0)
      def _():
        pltpu.make_async_copy(rb, r_sh.at[sb], sms).wait()

      compute(ch0, rc, rb, sb)
      pltpu.async_copy(rb, r_sh.at[sb], sms, add=True)

      @pl.when(j < NPAIR - 1)
      def _():
        issue_gather(ch0 + 2, rc, smr)
    return carry

  lax.fori_loop(0, NPAIR, pair_body, 0)
  pltpu.make_async_copy(rb0, r_sh.at[sb0], sms0).wait()
  pltpu.make_async_copy(rb1, r_sh.at[sb1], sms1).wait()

  pltpu.sync_copy(rec_h.at[se.at[pl.ds(NCHUNK * CH, L)]],
                  rc0.at[pl.ds(0, L)])
  pltpu.sync_copy(rec_h.at[de.at[pl.ds(NCHUNK * CH, L)]],
                  rc0.at[pl.ds(L, L)])
  seg, dist = edge_group(rc0, NCHUNK * CH, 0, L)
  tib[...] = seg
  for k in range(K):
    t = dist - _CENTERS[k]
    v = jnp.exp(t * t * (-10.0))
    col = jnp.full((L,), k, jnp.int32)
    plsc.store_scatter(rb0, [lane, col], v)
  pltpu.sync_copy(rb0.at[pl.ds(0, L)], r_sh.at[tib], add=True)

  plsc.subcore_barrier()

  pltpu.sync_copy(r_sh.at[pl.ds(s * ROWS_PT, ROWS_PT)],
                  out_h.at[c, pl.ds(s * ROWS_PT, ROWS_PT)])


@functools.lru_cache(maxsize=None)
def _build_sc_kernel():
  return functools.partial(
      pl.kernel,
      out_type=jax.ShapeDtypeStruct((NC, R_ROWS, K), jnp.float32),
      mesh=plsc.VectorSubcoreMesh(core_axis_name="c", subcore_axis_name="s"),
      compiler_params=pltpu.CompilerParams(
          needs_layout_passes=False, use_tc_tiling_on_sc=False),
      scratch_types=[
          pltpu.VMEM((EPT,), jnp.int32),
          pltpu.VMEM((EPT,), jnp.int32),
          pltpu.VMEM((2 * CH, L), jnp.float32),
          pltpu.VMEM((2 * CH, L), jnp.float32),
          pltpu.VMEM((CH, L), jnp.float32),
          pltpu.VMEM((CH, L), jnp.float32),
          pltpu.VMEM((CH,), jnp.int32),
          pltpu.VMEM((CH,), jnp.int32),
          pltpu.VMEM((L,), jnp.int32),
          pltpu.SemaphoreType.DMA,
          pltpu.SemaphoreType.DMA,
          pltpu.SemaphoreType.DMA,
          pltpu.SemaphoreType.DMA,
          pltpu.SemaphoreType.DMA,
          pltpu.VMEM_SHARED((R_ROWS, K), jnp.float32),
      ],
  )(_sc_edge_body)


def _atom_body(r_ref, zv_ref, wf_ref, we_ref, wu_ref, out_ref):
  cval = (lax.broadcasted_iota(jnp.int32, (NCLS, 1, F), 0) + 1
          ).astype(jnp.float32)
  h_tab = jnp.tanh(cval * we_ref[...])
  wbig = (h_tab * wf_ref[...]).reshape(NCLS * K, F)
  rsum = r_ref[0] + r_ref[1]
  agg = jnp.dot(rsum, wbig, preferred_element_type=jnp.float32)
  h0 = jnp.tanh(zv_ref[...] * we_ref[0])
  s_i = jnp.tanh(h0 + jnp.dot(agg, wu_ref[...],
                              preferred_element_type=jnp.float32))
  out_ref[...] = jnp.mean(s_i.reshape(N_CG, APC, F), axis=1)


def _stage2(r6, zv, w_filter, w_embed, w_update):
  full = lambda shape: pl.BlockSpec(shape, lambda: tuple(0 for _ in shape))
  return pl.pallas_call(
      _atom_body,
      in_specs=[
          full((NC, N_AT, NCLS * K)),
          full((N_AT, 1)),
          full((1, K, F)),
          full((1, F)),
          full((F, F)),
      ],
      out_specs=full((N_CG, F)),
      out_shape=jax.ShapeDtypeStruct((N_CG, F), jnp.float32),
  )(r6, zv, w_filter, w_embed, w_update)


BE = 2000
NBE = E_CG // BE


def _cg_body(cic_ref, cit_ref, cjc_ref, si_ref, cgp_ref,
             wmu1_ref, wmu2_ref, wsg1_ref, wsg2_ref, wcgf_ref, wv_ref,
             mu_ref, sg_ref, rx_ref, ry_ref, rz_ref,
             accx, accy, accz):
  step = pl.program_id(0)
  ohi = (cic_ref[...] == lax.broadcasted_iota(jnp.int32, (BE, N_CG), 1)
         ).astype(jnp.float32)
  ohj = (cjc_ref[...] == lax.broadcasted_iota(jnp.int32, (BE, N_CG), 1)
         ).astype(jnp.float32)
  ohit = (cit_ref[...].reshape(1, BE) ==
          lax.broadcasted_iota(jnp.int32, (N_CG, BE), 0)
          ).astype(jnp.float32)
  cgp = cgp_ref[...]
  gpi = jnp.dot(ohi, cgp, preferred_element_type=jnp.float32)
  gpj = jnp.dot(ohj, cgp, preferred_element_type=jnp.float32)
  dux = gpj[:, 0:1] - gpi[:, 0:1]
  duy = gpj[:, 1:2] - gpi[:, 1:2]
  duz = gpj[:, 2:3] - gpi[:, 2:3]
  dn = jnp.sqrt(dux * dux + duy * duy + duz * duz + 1e-8)
  ux = dux / dn
  uy = duy / dn
  uz = duz / dn
  cent = lax.broadcasted_iota(jnp.int32, (BE, K), 1).astype(jnp.float32) * (
      5.0 / (K - 1))
  t = dn - cent
  crbf = jnp.exp(t * t * (-10.0))
  sij = jnp.dot(ohj, si_ref[...], preferred_element_type=jnp.float32)
  cm = sij * jnp.dot(crbf, wcgf_ref[...], preferred_element_type=jnp.float32)
  w = jnp.dot(cm, wv_ref[...], preferred_element_type=jnp.float32)
  scx = jnp.dot(ohit, w * ux, preferred_element_type=jnp.float32)
  scy = jnp.dot(ohit, w * uy, preferred_element_type=jnp.float32)
  scz = jnp.dot(ohit, w * uz, preferred_element_type=jnp.float32)

  @pl.when(step == 0)
  def _():
    accx[...] = scx
    accy[...] = scy
    accz[...] = scz

  @pl.when(step != 0)
  def _():
    accx[...] = accx[...] + scx
    accy[...] = accy[...] + scy
    accz[...] = accz[...] + scz

  @pl.when(step == NBE - 1)
  def _():
    si = si_ref[...]
    mu_ref[...] = jnp.dot(
        jnp.tanh(jnp.dot(si, wmu1_ref[...], preferred_element_type=jnp.float32)),
        wmu2_ref[...], preferred_element_type=jnp.float32)
    logvar = jnp.dot(
        jnp.tanh(jnp.dot(si, wsg1_ref[...], preferred_element_type=jnp.float32)),
        wsg2_ref[...], preferred_element_type=jnp.float32)
    sg_ref[...] = 1e-12 + jnp.exp(logvar * 0.5)
    chmask = (lax.broadcasted_iota(jnp.int32, (N_CG, L), 1) < APC
              ).astype(jnp.float32)
    for acc, ref, col in ((accx, rx_ref, 0), (accy, ry_ref, 1), (accz, rz_ref, 2)):
      v = acc[...]
      offs = jnp.sum(v * chmask, axis=1, keepdims=True) * (1.0 / APC)
      ref[...] = v - offs + cgp_ref[:, col:col + 1]


def _stage3(cic, cit, cjc, s_i, cgp, w_mu1, w_mu2, w_sg1, w_sg2, w_cgf, w_vp):
  full = lambda shape: pl.BlockSpec(shape, lambda i: tuple(0 for _ in shape))
  return pl.pallas_call(
      _cg_body,
      grid=(NBE,),
      in_specs=[
          pl.BlockSpec((BE, 1), lambda i: (i, 0)),
          pl.BlockSpec((1, 1, BE), lambda i: (i, 0, 0)),
          pl.BlockSpec((BE, 1), lambda i: (i, 0)),
          full((N_CG, F)),
          full((N_CG, 8)),
          full((F, F)), full((F, F)), full((F, F)), full((F, F)),
          full((K, F)), full((F, L)),
      ],
      out_specs=[
          full((N_CG, F)), full((N_CG, F)),
          full((N_CG, L)), full((N_CG, L)), full((N_CG, L)),
      ],
      out_shape=[
          jax.ShapeDtypeStruct((N_CG, F), jnp.float32),
          jax.ShapeDtypeStruct((N_CG, F), jnp.float32),
          jax.ShapeDtypeStruct((N_CG, L), jnp.float32),
          jax.ShapeDtypeStruct((N_CG, L), jnp.float32),
          jax.ShapeDtypeStruct((N_CG, L), jnp.float32),
      ],
      scratch_shapes=[
          pltpu.VMEM((N_CG, L), jnp.float32),
          pltpu.VMEM((N_CG, L), jnp.float32),
          pltpu.VMEM((N_CG, L), jnp.float32),
      ],
  )(cic, cit, cjc, s_i, cgp, w_mu1, w_mu2, w_sg1, w_sg2, w_cgf, w_vp)


def kernel(nxyz, CG_nxyz, CG_mapping, nbr_list, CG_nbr_list, num_CGs,
           W_embed, W_filter, W_update, W_mu1, W_mu2, W_sg1, W_sg2,
           W_cgf, W_cgs, W_v):
  xyz = nxyz[:, 1:]
  rec = jnp.pad(jnp.concatenate([nxyz[:, 1:4], nxyz[:, 0:1]], axis=1),
                ((0, 0), (0, L - 4)))

  r_part = _build_sc_kernel()(rec, nbr_list[:, 0], nbr_list[:, 1])
  r6 = r_part.reshape(NC, N_AT, NCLS * K)

  s_i_cg = _stage2(r6, nxyz[:, 0:1],
                   W_filter.astype(jnp.float32).reshape(1, K, F),
                   W_embed.astype(jnp.float32), W_update.astype(jnp.float32))

  cic = jnp.asarray(CG_nbr_list[:, 0]).reshape(E_CG, 1)
  cit = cic.reshape(NBE, 1, BE)
  cjc = jnp.asarray(CG_nbr_list[:, 1]).reshape(E_CG, 1)
  cgp = jnp.pad(CG_nxyz[:, 1:], ((0, 0), (0, 5)))
  w_vp = jnp.pad(W_v.astype(jnp.float32), ((0, 0), (0, L - APC)))

  mu, sigma, rx, ry, rz = _stage3(
      cic, cit, cjc, s_i_cg, cgp,
      W_mu1.astype(jnp.float32), W_mu2.astype(jnp.float32),
      W_sg1.astype(jnp.float32), W_sg2.astype(jnp.float32),
      W_cgf.astype(jnp.float32), w_vp)

  xyz_recon = jnp.stack([
      rx[:, :APC].reshape(N_AT),
      ry[:, :APC].reshape(N_AT),
      rz[:, :APC].reshape(N_AT),
  ], axis=1)
  return (mu, sigma, xyz, xyz_recon)
